# baseline (speedup 1.0000x reference)
import functools

import jax
import jax.numpy as jnp
from jax import lax
from jax.experimental import pallas as pl
from jax.experimental.pallas import tpu as pltpu
from jax.experimental.pallas import tpu_sc as plsc

_CH = 8
_OFFS = (0, 2560, 5120, 7680)
_WIDS = (2560, 2560, 2560, 2304)


def _sc_gather_body(gp, tig_h, tjg_h, ti_h, tj_h, a01_h, a1_h, a012_h,
                    t01_h, t1_h, t012_h, x_h,
                    g01i, g01j, g1i, g1j, g012i, g012j, xi_o, xj_o,
                    big, bjg, bti, btj, b0, b1_, b2, b3, b4, b5, bxi, bxj,
                    sem):
    nc = 2
    wid = lax.axis_index("s") * nc + lax.axis_index("c")
    grp = wid // 4
    sid = wid % 4
    base = grp * gp
    pltpu.sync_copy(tig_h.at[pl.ds(base, gp)], big)
    pltpu.sync_copy(tjg_h.at[pl.ds(base, gp)], bjg)
    pltpu.sync_copy(ti_h.at[pl.ds(base, gp)], bti)
    pltpu.sync_copy(tj_h.at[pl.ds(base, gp)], btj)
    ys = (a01_h.bitcast(jnp.int32), a1_h.bitcast(jnp.int32),
          a012_h.bitcast(jnp.int32))
    yts = (t01_h.bitcast(jnp.int32), t1_h.bitcast(jnp.int32),
           t012_h.bitcast(jnp.int32))
    outs = (g01i, g01j, g1i, g1j, g012i, g012j)
    bufs = (b0, b1_, b2, b3, b4, b5)

    for s in range(4):
        @pl.when(sid == s)
        def _(s=s):
            off, w = _OFFS[s], _WIDS[s]
            ww = w + 128 if s == 3 else w

            def chunk(c, _):
                cb = pl.multiple_of(c * _CH, _CH)
                pb = pl.multiple_of(base + cb, _CH)
                idx = (big.at[pl.ds(cb, _CH)], bjg.at[pl.ds(cb, _CH)])
                cps = []
                for m in range(3):
                    for e in range(2):
                        cps.append(pltpu.async_copy(
                            ys[m].at[idx[e], pl.ds(off, w)],
                            bufs[2 * m + e].at[:, pl.ds(0, w)], sem))
                        if s == 3:
                            cps.append(pltpu.async_copy(
                                yts[m].at[idx[e]],
                                bufs[2 * m + e].at[:, pl.ds(w, 128)], sem))
                if s == 0:
                    cps.append(pltpu.async_copy(
                        x_h.at[bti.at[pl.ds(cb, _CH)]], bxi, sem))
                    cps.append(pltpu.async_copy(
                        x_h.at[btj.at[pl.ds(cb, _CH)]], bxj, sem))
                for cp in cps:
                    cp.wait()
                for m in range(6):
                    pltpu.sync_copy(
                        bufs[m].at[:, pl.ds(0, ww)],
                        outs[m].at[pl.ds(pb, _CH), pl.ds(off, ww)])
                if s == 0:
                    pltpu.sync_copy(bxi, xi_o.at[pl.ds(pb, _CH)])
                    pltpu.sync_copy(bxj, xj_o.at[pl.ds(pb, _CH)])
                return 0

            lax.fori_loop(0, gp // _CH, chunk, 0)


def _sc_gather(tig, tjg, ti, tj, a01b, a1b, a012b, t01, t1, t012, x):
    bsz = ti.shape[0]
    gp = bsz // 8
    d = x.shape[1]
    n2g = _OFFS[3] + _WIDS[3] + 128
    mesh = plsc.VectorSubcoreMesh(core_axis_name="c", subcore_axis_name="s")
    out_type = [jax.ShapeDtypeStruct((bsz, n2g), jnp.int32) for _ in range(6)]
    out_type += [jax.ShapeDtypeStruct((bsz, d), jnp.float32) for _ in range(2)]
    scratch = [pltpu.VMEM((gp,), jnp.int32) for _ in range(4)]
    scratch += [pltpu.VMEM((_CH, 2560), jnp.int32) for _ in range(6)]
    scratch += [pltpu.VMEM((_CH, d), jnp.float32) for _ in range(2)]
    scratch += [pltpu.SemaphoreType.DMA]
    return pl.kernel(
        functools.partial(_sc_gather_body, gp), mesh=mesh, out_type=out_type,
        scratch_types=scratch,
    )(tig, tjg, ti, tj, a01b, a1b, a012b, t01, t1, t012, x)


def _extract_spmm_body(g01i, g01j, g1i, g1j, g012i, g012j, shi_ref, shj_ref,
                       xi_ref, xj_ref, x_ref, wt_ref, b_ref, out_ref):
    bgl = g01i.shape[1]
    n2g = g01i.shape[2]
    d = x_ref.shape[1]
    shi = jnp.broadcast_to(shi_ref[0][:, 0:1], (bgl, n2g))
    shj = jnp.broadcast_to(shj_ref[0][:, 0:1], (bgl, n2g))
    c01 = lax.shift_right_logical(g01i[0], shi) & \
        lax.shift_right_logical(g01j[0], shj)
    c1 = lax.shift_right_logical(g1i[0], shi) & \
        lax.shift_right_logical(g1j[0], shj)
    c012 = lax.shift_right_logical(g012i[0], shi) & \
        lax.shift_right_logical(g012j[0], shj)
    m0 = (c01 & ~c1 & 1).astype(jnp.float32)
    m1 = (c1 & 1).astype(jnp.float32)
    m2 = (c012 & ~c01 & 1).astype(jnp.float32)
    xij = xi_ref[0] * xj_ref[0]
    acc = jnp.dot(xij, wt_ref[0:d, :], preferred_element_type=jnp.float32)
    for k, mk in enumerate((m0, m1, m2)):
        t = jnp.dot(mk, x_ref[...], preferred_element_type=jnp.float32)
        acc = acc + jnp.dot(t, wt_ref[(k + 1) * d:(k + 2) * d, :],
                            preferred_element_type=jnp.float32)
    out_ref[0] = acc + b_ref[0]


@jax.jit
def kernel(x, adj_0_1, adj_1, adj_0_1_2, tar_ei, W, b):
    n, d = x.shape
    bsz = tar_ei.shape[1]
    out_dim = W.shape[0]
    ti = tar_ei[0].astype(jnp.int32)
    tj = tar_ei[1].astype(jnp.int32)
    tig = ti // 4
    tjg = tj // 4
    shi = jnp.broadcast_to(((ti % 4) * 8)[:, None], (bsz, 128))
    shj = jnp.broadcast_to(((tj % 4) * 8)[:, None], (bsz, 128))

    mw4 = n // 128 * 128
    a01b = adj_0_1.view(jnp.int8)
    a1b = adj_1.view(jnp.int8)
    a012b = adj_0_1_2.view(jnp.int8)

    def tail(a):
        return jnp.pad(a[:, mw4:], ((0, 0), (0, 128 - (n - mw4))))

    t01m, t1m, t012m = tail(a01b), tail(a1b), tail(a012b)
    half = bsz // 2
    gath = [
        _sc_gather(tig[h * half:(h + 1) * half], tjg[h * half:(h + 1) * half],
                   ti[h * half:(h + 1) * half], tj[h * half:(h + 1) * half],
                   a01b, a1b, a012b, t01m, t1m, t012m, x)
        for h in range(2)
    ]
    n2g = gath[0][0].shape[1]
    xpad = jnp.pad(x, ((0, n2g - n), (0, 0)))

    bg = 64
    nb = half // bg
    word_spec = pl.BlockSpec((1, bg, n2g), lambda i: (i, 0, 0))
    sh_spec = pl.BlockSpec((1, bg, 128), lambda i: (i, 0, 0))
    xrow_spec = pl.BlockSpec((1, bg, d), lambda i: (i, 0, 0))
    x_spec = pl.BlockSpec((n2g, d), lambda i: (0, 0))
    wt_spec = pl.BlockSpec((4 * d, out_dim), lambda i: (0, 0))
    b_spec = pl.BlockSpec((1, out_dim), lambda i: (0, 0))

    halves = []
    for h in range(2):
        gathered = gath[h]
        words = [wv.reshape(nb, bg, n2g) for wv in gathered[:6]]
        xir = gathered[6].reshape(nb, bg, d)
        xjr = gathered[7].reshape(nb, bg, d)
        shir = shi[h * half:(h + 1) * half].reshape(nb, bg, 128)
        shjr = shj[h * half:(h + 1) * half].reshape(nb, bg, 128)
        out = pl.pallas_call(
            _extract_spmm_body,
            grid=(nb,),
            in_specs=[word_spec] * 6 + [sh_spec, sh_spec, xrow_spec,
                                        xrow_spec, x_spec, wt_spec, b_spec],
            out_specs=pl.BlockSpec((1, bg, out_dim), lambda i: (i, 0, 0)),
            out_shape=jax.ShapeDtypeStruct((nb, bg, out_dim), jnp.float32),
        )(*words, shir, shjr, xir, xjr, xpad, W.T, b.reshape(1, out_dim))
        halves.append(out.reshape(half, out_dim))

    return jnp.concatenate(halves, axis=0)


# speedup vs baseline: 1.0235x; 1.0235x over previous
import functools

import jax
import jax.numpy as jnp
from jax import lax
from jax.experimental import pallas as pl
from jax.experimental.pallas import tpu as pltpu
from jax.experimental.pallas import tpu_sc as plsc

_GP = 128
_CH = 8
_OFFS = (0, 2560, 5120, 7680)
_WIDS = (2560, 2560, 2560, 2304)


def _sc_gather_body(tig_h, tjg_h, ti_h, tj_h, a01_h, a1_h, a012_h,
                    t01_h, t1_h, t012_h, x_h,
                    g01i, g01j, g1i, g1j, g012i, g012j, xi_o, xj_o,
                    big, bjg, bti, btj, b0, b1_, b2, b3, b4, b5, bxi, bxj,
                    sem):
    nc = 2
    wid = lax.axis_index("s") * nc + lax.axis_index("c")
    grp = wid // 4
    sid = wid % 4
    base = grp * _GP
    pltpu.sync_copy(tig_h.at[pl.ds(base, _GP)], big)
    pltpu.sync_copy(tjg_h.at[pl.ds(base, _GP)], bjg)
    pltpu.sync_copy(ti_h.at[pl.ds(base, _GP)], bti)
    pltpu.sync_copy(tj_h.at[pl.ds(base, _GP)], btj)
    ys = (a01_h.bitcast(jnp.int32), a1_h.bitcast(jnp.int32),
          a012_h.bitcast(jnp.int32))
    yts = (t01_h.bitcast(jnp.int32), t1_h.bitcast(jnp.int32),
           t012_h.bitcast(jnp.int32))
    outs = (g01i, g01j, g1i, g1j, g012i, g012j)
    bufs = (b0, b1_, b2, b3, b4, b5)

    for s in range(4):
        @pl.when(sid == s)
        def _(s=s):
            off, w = _OFFS[s], _WIDS[s]
            ww = w + 128 if s == 3 else w

            def chunk(c, _):
                cb = pl.multiple_of(c * _CH, _CH)
                pb = pl.multiple_of(base + cb, _CH)
                idx = (big.at[pl.ds(cb, _CH)], bjg.at[pl.ds(cb, _CH)])
                cps = []
                for m in range(3):
                    for e in range(2):
                        cps.append(pltpu.async_copy(
                            ys[m].at[idx[e], pl.ds(off, w)],
                            bufs[2 * m + e].at[:, pl.ds(0, w)], sem))
                        if s == 3:
                            cps.append(pltpu.async_copy(
                                yts[m].at[idx[e]],
                                bufs[2 * m + e].at[:, pl.ds(w, 128)], sem))
                if s == 0:
                    cps.append(pltpu.async_copy(
                        x_h.at[bti.at[pl.ds(cb, _CH)]], bxi, sem))
                    cps.append(pltpu.async_copy(
                        x_h.at[btj.at[pl.ds(cb, _CH)]], bxj, sem))
                for cp in cps:
                    cp.wait()
                for m in range(6):
                    pltpu.sync_copy(
                        bufs[m].at[:, pl.ds(0, ww)],
                        outs[m].at[pl.ds(pb, _CH), pl.ds(off, ww)])
                if s == 0:
                    pltpu.sync_copy(bxi, xi_o.at[pl.ds(pb, _CH)])
                    pltpu.sync_copy(bxj, xj_o.at[pl.ds(pb, _CH)])
                return 0

            lax.fori_loop(0, _GP // _CH, chunk, 0)


def _sc_gather(tig, tjg, ti, tj, a01b, a1b, a012b, t01, t1, t012, x):
    bsz = ti.shape[0]
    d = x.shape[1]
    n2g = _OFFS[3] + _WIDS[3] + 128
    mesh = plsc.VectorSubcoreMesh(core_axis_name="c", subcore_axis_name="s")
    out_type = [jax.ShapeDtypeStruct((bsz, n2g), jnp.int32) for _ in range(6)]
    out_type += [jax.ShapeDtypeStruct((bsz, d), jnp.float32) for _ in range(2)]
    scratch = [pltpu.VMEM((_GP,), jnp.int32) for _ in range(4)]
    scratch += [pltpu.VMEM((_CH, 2560), jnp.int32) for _ in range(6)]
    scratch += [pltpu.VMEM((_CH, d), jnp.float32) for _ in range(2)]
    scratch += [pltpu.SemaphoreType.DMA]
    return pl.kernel(
        _sc_gather_body, mesh=mesh, out_type=out_type, scratch_types=scratch,
    )(tig, tjg, ti, tj, a01b, a1b, a012b, t01, t1, t012, x)


def _extract_spmm_body(g01i, g01j, g1i, g1j, g012i, g012j, shi_ref, shj_ref,
                       xi_ref, xj_ref, x_ref, wt_ref, b_ref, out_ref):
    bgl = g01i.shape[1]
    n2g = g01i.shape[2]
    d = x_ref.shape[1]
    shi = jnp.broadcast_to(shi_ref[0][:, 0:1], (bgl, n2g))
    shj = jnp.broadcast_to(shj_ref[0][:, 0:1], (bgl, n2g))
    c01 = lax.shift_right_logical(g01i[0], shi) & \
        lax.shift_right_logical(g01j[0], shj)
    c1 = lax.shift_right_logical(g1i[0], shi) & \
        lax.shift_right_logical(g1j[0], shj)
    c012 = lax.shift_right_logical(g012i[0], shi) & \
        lax.shift_right_logical(g012j[0], shj)
    m0 = (c01 & ~c1 & 1).astype(jnp.float32)
    m1 = (c1 & 1).astype(jnp.float32)
    m2 = (c012 & ~c01 & 1).astype(jnp.float32)
    xij = xi_ref[0] * xj_ref[0]
    acc = jnp.dot(xij, wt_ref[0:d, :], preferred_element_type=jnp.float32)
    for k, mk in enumerate((m0, m1, m2)):
        t = jnp.dot(mk, x_ref[...], preferred_element_type=jnp.float32)
        acc = acc + jnp.dot(t, wt_ref[(k + 1) * d:(k + 2) * d, :],
                            preferred_element_type=jnp.float32)
    out_ref[0] = acc + b_ref[0]


@jax.jit
def kernel(x, adj_0_1, adj_1, adj_0_1_2, tar_ei, W, b):
    n, d = x.shape
    bsz = tar_ei.shape[1]
    out_dim = W.shape[0]
    ti = tar_ei[0].astype(jnp.int32)
    tj = tar_ei[1].astype(jnp.int32)
    tig = ti // 4
    tjg = tj // 4
    shi = jnp.broadcast_to(((ti % 4) * 8)[:, None], (bsz, 128))
    shj = jnp.broadcast_to(((tj % 4) * 8)[:, None], (bsz, 128))

    mw4 = n // 128 * 128
    a01b = adj_0_1.view(jnp.int8)
    a1b = adj_1.view(jnp.int8)
    a012b = adj_0_1_2.view(jnp.int8)

    def tail(a):
        return jnp.pad(a[:, mw4:], ((0, 0), (0, 128 - (n - mw4))))

    gathered = _sc_gather(tig, tjg, ti, tj, a01b, a1b, a012b,
                          tail(a01b), tail(a1b), tail(a012b), x)
    words = gathered[:6]
    xi, xj = gathered[6], gathered[7]
    n2g = words[0].shape[1]
    xpad = jnp.pad(x, ((0, n2g - n), (0, 0)))

    bg = 64
    nb = bsz // bg
    words = [wv.reshape(nb, bg, n2g) for wv in words]
    xir = xi.reshape(nb, bg, d)
    xjr = xj.reshape(nb, bg, d)
    shir = shi.reshape(nb, bg, 128)
    shjr = shj.reshape(nb, bg, 128)

    word_spec = pl.BlockSpec((1, bg, n2g), lambda i: (i, 0, 0))
    sh_spec = pl.BlockSpec((1, bg, 128), lambda i: (i, 0, 0))
    xrow_spec = pl.BlockSpec((1, bg, d), lambda i: (i, 0, 0))
    x_spec = pl.BlockSpec((n2g, d), lambda i: (0, 0))
    wt_spec = pl.BlockSpec((4 * d, out_dim), lambda i: (0, 0))
    b_spec = pl.BlockSpec((1, out_dim), lambda i: (0, 0))

    out = pl.pallas_call(
        _extract_spmm_body,
        grid=(nb,),
        in_specs=[word_spec] * 6 + [sh_spec, sh_spec, xrow_spec, xrow_spec,
                                    x_spec, wt_spec, b_spec],
        out_specs=pl.BlockSpec((1, bg, out_dim), lambda i: (i, 0, 0)),
        out_shape=jax.ShapeDtypeStruct((nb, bg, out_dim), jnp.float32),
    )(*words, shir, shjr, xir, xjr, xpad, W.T, b.reshape(1, out_dim))

    return out.reshape(bsz, out_dim)
